# baseline (speedup 1.0000x reference)
import functools

import jax
import jax.numpy as jnp
from jax import lax
from jax.experimental import pallas as pl
from jax.experimental.pallas import tpu as pltpu
from jax.experimental.pallas import tpu_sc as plsc

N = 50000
E = 800000
H = 64
G = 128

NC = 2
NS = 16

PADN = 50048
STRIPE = PADN // NS
WB_LAST = N - (NS - 1) * STRIPE

CB = 80
GQ = 5
GE = GQ * CB
ZB = 392

_F32 = jnp.float32
_I32 = jnp.int32


def _make_segsum(D, mode):
    mesh = plsc.VectorSubcoreMesh(core_axis_name="c", subcore_axis_name="s")
    nchunk = (E if mode == "cols" else E // 2) // CB
    ngrp_sc = nchunk // GQ

    @functools.partial(
        pl.kernel,
        mesh=mesh,
        out_type=jax.ShapeDtypeStruct((NC, N, D), _F32),
        scratch_types=[
            pltpu.VMEM((2, GE), _I32),
            pltpu.VMEM((2 * GQ, CB), _I32),
            [pltpu.VMEM((CB,), _I32) for _ in range(GQ)],
            [pltpu.VMEM((CB, D), _F32) for _ in range(GQ)],
            pltpu.VMEM((ZB, D), _F32),
            pltpu.VMEM_SHARED((PADN, D), _F32),
            [pltpu.SemaphoreType.DMA for _ in range(GQ)],
            pltpu.SemaphoreType.DMA,
            pltpu.SemaphoreType.DMA,
        ],
        compiler_params=pltpu.CompilerParams(use_tc_tiling_on_sc=False),
    )
    def seg(h_hbm, src_hbm, dst2_hbm, out_hbm, sg, dg, gl, rows, zb, acc,
            gsem, ssem, isem):
        cid = lax.axis_index("c")
        sid = lax.axis_index("s")

        z16 = jnp.zeros((16,), _F32)

        def zfill(i, carry):
            for t in range(D // 16):
                zb[i, pl.ds(t * 16, 16)] = z16
            return carry

        lax.fori_loop(0, ZB, zfill, 0)
        zr0 = sid * STRIPE
        for q in range(STRIPE // ZB):
            pltpu.sync_copy(zb, acc.at[pl.ds(zr0 + q * ZB, ZB)])
        zrem = STRIPE % ZB
        if zrem:
            pltpu.sync_copy(zb.at[pl.ds(0, zrem)],
                            acc.at[pl.ds(zr0 + (STRIPE // ZB) * ZB, zrem)])
        plsc.subcore_barrier()

        if mode == "cols":
            ngrp_t = ngrp_sc // NS
            grp0 = sid * ngrp_t
            gstep = 1
        else:
            ngrp_t = (ngrp_sc - sid + NS - 1) // NS
            grp0 = cid * ngrp_sc + sid
            gstep = NS

        def fetch_idx(j, parity):
            cbase = (grp0 + j * gstep) * GQ
            pltpu.async_copy(src_hbm.at[pl.ds(cbase * CB, GE)], sg.at[parity],
                             isem)
            pltpu.async_copy(dst2_hbm.at[pl.ds(cbase, GQ)],
                             dg.at[pl.ds(parity * GQ, GQ)], isem)

        fetch_idx(0, 0)

        def group(j, carry):
            par = lax.rem(j, 2)
            pltpu.make_async_copy(src_hbm.at[pl.ds(0, GE)], sg.at[par],
                                  isem).wait()
            pltpu.make_async_copy(dst2_hbm.at[pl.ds(0, GQ)],
                                  dg.at[pl.ds(0, GQ)], isem).wait()
            @pl.when(j > 0)
            def _():
                for q in range(GQ):
                    pltpu.make_async_copy(
                        h_hbm.at[pl.ds(0, CB)], rows[q], ssem).wait()

            @pl.when(j + 1 < ngrp_t)
            def _():
                fetch_idx(j + 1, 1 - par)

            gd = []
            for q in range(GQ):
                for t in range(CB // 16):
                    sv = sg[par, pl.ds(q * CB + t * 16, 16)]
                    if mode == "cols":
                        gl[q][pl.ds(t * 16, 16)] = sv * 2 + cid
                    else:
                        gl[q][pl.ds(t * 16, 16)] = sv
                gd.append(pltpu.async_copy(h_hbm.at[gl[q]], rows[q], gsem[q]))
            for q in range(GQ):
                gd[q].wait()
                pltpu.async_copy(rows[q], acc.at[dg.at[par * GQ + q]], ssem,
                                 add=True)
            return carry

        lax.fori_loop(0, ngrp_t, group, 0)
        for q in range(GQ):
            pltpu.make_async_copy(h_hbm.at[pl.ds(0, CB)], rows[q], ssem).wait()
        plsc.subcore_barrier()

        r0 = sid * STRIPE

        @pl.when(sid < NS - 1)
        def _():
            pltpu.sync_copy(acc.at[pl.ds(r0, STRIPE)],
                            out_hbm.at[cid, pl.ds(r0, STRIPE)])

        @pl.when(sid == NS - 1)
        def _():
            pltpu.sync_copy(acc.at[pl.ds(r0, WB_LAST)],
                            out_hbm.at[cid, pl.ds(r0, WB_LAST)])

    return seg


_segsum16 = _make_segsum(16, "edges")
_segsum32 = _make_segsum(32, "cols")


PR = 1568
PA = 784
PB_LAST = N - 31 * PR - PA


def _pool_rows(rows, bat, mx, sm, cn, lane0, nrows):
    def row(r, carry):
        seg = bat[pl.ds(r, 16)][0]
        for t in range(4):
            v = rows[r, pl.ds(t * 16, 16)]
            mx[seg, pl.ds(t * 16, 16)] = jnp.maximum(mx[seg, pl.ds(t * 16, 16)], v)
            sm[seg, pl.ds(t * 16, 16)] = sm[seg, pl.ds(t * 16, 16)] + v
        cn[seg, pl.ds(0, 16)] = cn[seg, pl.ds(0, 16)] + lane0
        return carry

    lax.fori_loop(0, nrows, row, 0)


def _make_pool():
    mesh = plsc.VectorSubcoreMesh(core_axis_name="c", subcore_axis_name="s")

    @functools.partial(
        pl.kernel,
        mesh=mesh,
        out_type=(
            jax.ShapeDtypeStruct((NC * NS, G, H), _F32),
            jax.ShapeDtypeStruct((NC * NS, G, H), _F32),
            jax.ShapeDtypeStruct((NC * NS, G, 16), _F32),
        ),
        scratch_types=[
            pltpu.VMEM((PA, H), _F32),
            pltpu.VMEM((PA + 16,), _I32),
            pltpu.VMEM((G, H), _F32),
            pltpu.VMEM((G, H), _F32),
            pltpu.VMEM((G, 16), _F32),
        ],
        compiler_params=pltpu.CompilerParams(use_tc_tiling_on_sc=False),
    )
    def pool(h_hbm, bat_hbm, omax, osum, ocnt, rows, bat, mx, sm, cn):
        cid = lax.axis_index("c")
        sid = lax.axis_index("s")
        wid = sid * NC + cid
        r0 = wid * PR

        ninf = jnp.full((16,), -jnp.inf, _F32)
        z16 = jnp.zeros((16,), _F32)
        iota = lax.iota(_I32, 16)
        lane0 = jnp.where(iota == 0, 1.0, 0.0).astype(_F32)

        def init(i, carry):
            for t in range(4):
                mx[i, pl.ds(t * 16, 16)] = ninf
                sm[i, pl.ds(t * 16, 16)] = z16
            cn[i, pl.ds(0, 16)] = z16
            return carry

        lax.fori_loop(0, G, init, 0)

        pltpu.sync_copy(h_hbm.at[pl.ds(r0, PA)], rows)
        pltpu.sync_copy(bat_hbm.at[pl.ds(r0, PA)], bat.at[pl.ds(0, PA)])
        _pool_rows(rows, bat, mx, sm, cn, lane0, PA)

        @pl.when(wid < NC * NS - 1)
        def _():
            pltpu.sync_copy(h_hbm.at[pl.ds(r0 + PA, PA)], rows)
            pltpu.sync_copy(bat_hbm.at[pl.ds(r0 + PA, PA)], bat.at[pl.ds(0, PA)])
            _pool_rows(rows, bat, mx, sm, cn, lane0, PA)

        @pl.when(wid == NC * NS - 1)
        def _():
            pltpu.sync_copy(h_hbm.at[pl.ds(r0 + PA, PB_LAST)],
                            rows.at[pl.ds(0, PB_LAST)])
            pltpu.sync_copy(bat_hbm.at[pl.ds(r0 + PA, PB_LAST)],
                            bat.at[pl.ds(0, PB_LAST)])
            _pool_rows(rows, bat, mx, sm, cn, lane0, PB_LAST)

        pltpu.sync_copy(mx, omax.at[wid])
        pltpu.sync_copy(sm, osum.at[wid])
        pltpu.sync_copy(cn, ocnt.at[wid])

    return pool


_pool = _make_pool()


R = 1000
NB = N // R


def _make_layer_body(combine):
    def _body(s0_ref, s1_ref, c_ref, hp_ref, wl_ref, bl_ref, wr_ref, g_ref,
              be_ref, h_ref, st_ref):
        p = pl.program_id(0)
        i = pl.program_id(1)
        inv = 1.0 / jnp.maximum(c_ref[...], 1.0)
        if combine == "sum":
            s = s0_ref[0] + s1_ref[0]
        else:
            s = jnp.concatenate([s0_ref[0], s1_ref[0]], axis=1)
        a = s * inv
        y = (jnp.dot(a, wl_ref[...], preferred_element_type=_F32)
             + bl_ref[...]
             + jnp.dot(hp_ref[...], wr_ref[...], preferred_element_type=_F32))

        @pl.when(p == 0)
        def _():
            @pl.when(i == 0)
            def _():
                st_ref[...] = jnp.zeros((8, H), _F32)

            su = jnp.sum(y, axis=0)
            sq = jnp.sum(y * y, axis=0)
            upd = jnp.concatenate(
                [su[None, :], sq[None, :], jnp.zeros((6, H), _F32)], axis=0)
            st_ref[...] = st_ref[...] + upd
            h_ref[...] = y

        @pl.when(p == 1)
        def _():
            st = st_ref[...]
            mean = st[0:1, :] * (1.0 / N)
            ex2 = st[1:2, :] * (1.0 / N)
            var = ex2 - mean * mean
            rstd = lax.rsqrt(var + 1e-5)
            h_ref[...] = jnp.maximum(
                (y - mean) * (rstd * g_ref[...]) + be_ref[...], 0.0)

    return _body


def _layer_tc(s01, cnt, hp, Wl, bl, Wr, g, be, combine):
    K = s01.shape[2]
    KW = Wl.shape[0]
    K2 = hp.shape[1]
    return pl.pallas_call(
        _make_layer_body(combine),
        grid=(2, NB),
        in_specs=[
            pl.BlockSpec((1, R, K), lambda p, i: (0, i, 0)),
            pl.BlockSpec((1, R, K), lambda p, i: (1, i, 0)),
            pl.BlockSpec((R, 1), lambda p, i: (i, 0)),
            pl.BlockSpec((R, K2), lambda p, i: (i, 0)),
            pl.BlockSpec((KW, H), lambda p, i: (0, 0)),
            pl.BlockSpec((1, H), lambda p, i: (0, 0)),
            pl.BlockSpec((K2, H), lambda p, i: (0, 0)),
            pl.BlockSpec((1, H), lambda p, i: (0, 0)),
            pl.BlockSpec((1, H), lambda p, i: (0, 0)),
        ],
        out_specs=pl.BlockSpec((R, H), lambda p, i: (i, 0)),
        out_shape=jax.ShapeDtypeStruct((N, H), _F32),
        scratch_shapes=[pltpu.VMEM((8, H), _F32)],
    )(s01, s01, cnt, hp, Wl, bl, Wr, g, be)


def _head_body(pm_ref, ps_ref, pc_ref, w1_ref, b1_ref, w2_ref, b2_ref, o_ref):
    mx = jnp.max(pm_ref[...], axis=0)
    sm = jnp.sum(ps_ref[...], axis=0)
    cnt = jnp.sum(pc_ref[...], axis=0)[:, 0:1]
    mean = sm / jnp.maximum(cnt, 1.0)
    z = jnp.concatenate([mx, mean], axis=1)
    r = jnp.maximum(
        jnp.dot(z, w1_ref[...], preferred_element_type=_F32) + b1_ref[...], 0.0)
    o_ref[...] = jnp.dot(r, w2_ref[...], preferred_element_type=_F32) + b2_ref[...]


def _head(pmax, psum, pcnt, W1, b1, W2, b2):
    return pl.pallas_call(
        _head_body,
        out_shape=jax.ShapeDtypeStruct((G, 2), _F32),
    )(pmax, psum, pcnt, W1, b1, W2, b2)


def kernel(x, edge_index, batch, Wl1, bl1, Wr1, g1, be1, Wl2, bl2, Wr2, g2,
           be2, Wl3, bl3, Wr3, g3, be3, W_lin1, b_lin1, W_lin2, b_lin2):
    src = edge_index[0]
    dst = edge_index[1]
    dst2 = dst.reshape(E // CB, CB)

    x16 = jnp.concatenate(
        [x, jnp.ones((N, 1), _F32), jnp.zeros((N, 9), _F32)], axis=1)
    Wl1p = jnp.zeros((16, H), _F32).at[:6].set(Wl1)
    Wr1p = jnp.zeros((16, H), _F32).at[:6].set(Wr1)

    s1 = _segsum16(x16, src, dst2)
    cnt = s1[0, :, 6:7] + s1[1, :, 6:7]

    h1 = _layer_tc(s1, cnt, x16, Wl1p, bl1.reshape(1, H), Wr1p,
                   g1.reshape(1, H), be1.reshape(1, H), "sum")

    s2 = _segsum32(h1.reshape(2 * N, H // 2), src, dst2)
    h2 = _layer_tc(s2, cnt, h1, Wl2, bl2.reshape(1, H), Wr2,
                   g2.reshape(1, H), be2.reshape(1, H), "cat")

    s3 = _segsum32(h2.reshape(2 * N, H // 2), src, dst2)
    h3 = _layer_tc(s3, cnt, h2, Wl3, bl3.reshape(1, H), Wr3,
                   g3.reshape(1, H), be3.reshape(1, H), "cat")

    pmax, psum, pcnt = _pool(h3, batch)
    out = _head(pmax, psum, pcnt, W_lin1, b_lin1.reshape(1, H),
                W_lin2, b_lin2.reshape(1, 2))
    return out


# speedup vs baseline: 1.1628x; 1.1628x over previous
import functools

import jax
import jax.numpy as jnp
from jax import lax
from jax.experimental import pallas as pl
from jax.experimental.pallas import tpu as pltpu
from jax.experimental.pallas import tpu_sc as plsc

N = 50000
E = 800000
H = 64
G = 128

NC = 2
NS = 16

PADN = 50048
STRIPE = PADN // NS
WB_LAST = N - (NS - 1) * STRIPE

CB = 80
GQ = 5
GE = GQ * CB
ZB = 392

_F32 = jnp.float32
_I32 = jnp.int32


def _make_segsum(D, mode):
    mesh = plsc.VectorSubcoreMesh(core_axis_name="c", subcore_axis_name="s")
    nchunk = (E if mode == "cols" else E // 2) // CB
    ngrp_sc = nchunk // GQ

    @functools.partial(
        pl.kernel,
        mesh=mesh,
        out_type=jax.ShapeDtypeStruct((N, NC * D), _F32),
        scratch_types=[
            pltpu.VMEM((2, GE), _I32),
            pltpu.VMEM((2 * GQ, CB), _I32),
            [pltpu.VMEM((CB,), _I32) for _ in range(GQ)],
            [pltpu.VMEM((CB, D), _F32) for _ in range(GQ)],
            pltpu.VMEM((ZB, D), _F32),
            pltpu.VMEM_SHARED((PADN, D), _F32),
            [pltpu.SemaphoreType.DMA for _ in range(GQ)],
            pltpu.SemaphoreType.DMA,
            pltpu.SemaphoreType.DMA,
        ],
        compiler_params=pltpu.CompilerParams(use_tc_tiling_on_sc=False),
    )
    def seg(h_hbm, src_hbm, dst2_hbm, out_hbm, sg, dg, gl, rows, zb, acc,
            gsem, ssem, isem):
        cid = lax.axis_index("c")
        sid = lax.axis_index("s")

        z16 = jnp.zeros((16,), _F32)

        def zfill(i, carry):
            for t in range(D // 16):
                zb[i, pl.ds(t * 16, 16)] = z16
            return carry

        lax.fori_loop(0, ZB, zfill, 0)
        zr0 = sid * STRIPE
        for q in range(STRIPE // ZB):
            pltpu.sync_copy(zb, acc.at[pl.ds(zr0 + q * ZB, ZB)])
        zrem = STRIPE % ZB
        if zrem:
            pltpu.sync_copy(zb.at[pl.ds(0, zrem)],
                            acc.at[pl.ds(zr0 + (STRIPE // ZB) * ZB, zrem)])
        plsc.subcore_barrier()

        if mode == "cols":
            ngrp_t = ngrp_sc // NS
            grp0 = sid * ngrp_t
            gstep = 1
        else:
            ngrp_t = (ngrp_sc - sid + NS - 1) // NS
            grp0 = cid * ngrp_sc + sid
            gstep = NS

        def fetch_idx(j, parity):
            cbase = (grp0 + j * gstep) * GQ
            pltpu.async_copy(src_hbm.at[pl.ds(cbase * CB, GE)], sg.at[parity],
                             isem)
            pltpu.async_copy(dst2_hbm.at[pl.ds(cbase, GQ)],
                             dg.at[pl.ds(parity * GQ, GQ)], isem)

        fetch_idx(0, 0)

        def group(j, carry):
            par = lax.rem(j, 2)
            pltpu.make_async_copy(src_hbm.at[pl.ds(0, GE)], sg.at[par],
                                  isem).wait()
            pltpu.make_async_copy(dst2_hbm.at[pl.ds(0, GQ)],
                                  dg.at[pl.ds(0, GQ)], isem).wait()
            @pl.when(j > 0)
            def _():
                for q in range(GQ):
                    pltpu.make_async_copy(
                        h_hbm.at[pl.ds(0, CB)], rows[q], ssem).wait()

            @pl.when(j + 1 < ngrp_t)
            def _():
                fetch_idx(j + 1, 1 - par)

            gd = []
            for q in range(GQ):
                for t in range(CB // 16):
                    sv = sg[par, pl.ds(q * CB + t * 16, 16)]
                    if mode == "cols":
                        gl[q][pl.ds(t * 16, 16)] = sv * 2 + cid
                    else:
                        gl[q][pl.ds(t * 16, 16)] = sv
                gd.append(pltpu.async_copy(h_hbm.at[gl[q]], rows[q], gsem[q]))
            for q in range(GQ):
                gd[q].wait()
                pltpu.async_copy(rows[q], acc.at[dg.at[par * GQ + q]], ssem,
                                 add=True)
            return carry

        lax.fori_loop(0, ngrp_t, group, 0)
        for q in range(GQ):
            pltpu.make_async_copy(h_hbm.at[pl.ds(0, CB)], rows[q], ssem).wait()
        plsc.subcore_barrier()

        r0 = sid * STRIPE

        @pl.when(sid < NS - 1)
        def _():
            pltpu.sync_copy(acc.at[pl.ds(r0, STRIPE)],
                            out_hbm.at[pl.ds(r0, STRIPE),
                                       pl.ds(cid * D, D)])

        @pl.when(sid == NS - 1)
        def _():
            pltpu.sync_copy(acc.at[pl.ds(r0, WB_LAST)],
                            out_hbm.at[pl.ds(r0, WB_LAST),
                                       pl.ds(cid * D, D)])

    return seg


_segsum16 = _make_segsum(16, "edges")
_segsum32 = _make_segsum(32, "cols")


PR = 1568
PA = 784
PB_LAST = N - 31 * PR - PA


def _pool_rows(rows, bat, mx, sm, cn, lane0, nrows):
    def row(r, carry):
        seg = bat[pl.ds(r, 16)][0]
        for t in range(4):
            v = rows[r, pl.ds(t * 16, 16)]
            mx[seg, pl.ds(t * 16, 16)] = jnp.maximum(mx[seg, pl.ds(t * 16, 16)], v)
            sm[seg, pl.ds(t * 16, 16)] = sm[seg, pl.ds(t * 16, 16)] + v
        cn[seg, pl.ds(0, 16)] = cn[seg, pl.ds(0, 16)] + lane0
        return carry

    lax.fori_loop(0, nrows, row, 0)


def _make_pool():
    mesh = plsc.VectorSubcoreMesh(core_axis_name="c", subcore_axis_name="s")

    @functools.partial(
        pl.kernel,
        mesh=mesh,
        out_type=(
            jax.ShapeDtypeStruct((NC * NS, G, H), _F32),
            jax.ShapeDtypeStruct((NC * NS, G, H), _F32),
            jax.ShapeDtypeStruct((NC * NS, G, 16), _F32),
        ),
        scratch_types=[
            pltpu.VMEM((PA, H), _F32),
            pltpu.VMEM((PA + 16,), _I32),
            pltpu.VMEM((G, H), _F32),
            pltpu.VMEM((G, H), _F32),
            pltpu.VMEM((G, 16), _F32),
        ],
        compiler_params=pltpu.CompilerParams(use_tc_tiling_on_sc=False),
    )
    def pool(h_hbm, bat_hbm, omax, osum, ocnt, rows, bat, mx, sm, cn):
        cid = lax.axis_index("c")
        sid = lax.axis_index("s")
        wid = sid * NC + cid
        r0 = wid * PR

        ninf = jnp.full((16,), -jnp.inf, _F32)
        z16 = jnp.zeros((16,), _F32)
        iota = lax.iota(_I32, 16)
        lane0 = jnp.where(iota == 0, 1.0, 0.0).astype(_F32)

        def init(i, carry):
            for t in range(4):
                mx[i, pl.ds(t * 16, 16)] = ninf
                sm[i, pl.ds(t * 16, 16)] = z16
            cn[i, pl.ds(0, 16)] = z16
            return carry

        lax.fori_loop(0, G, init, 0)

        pltpu.sync_copy(h_hbm.at[pl.ds(r0, PA)], rows)
        pltpu.sync_copy(bat_hbm.at[pl.ds(r0, PA)], bat.at[pl.ds(0, PA)])
        _pool_rows(rows, bat, mx, sm, cn, lane0, PA)

        @pl.when(wid < NC * NS - 1)
        def _():
            pltpu.sync_copy(h_hbm.at[pl.ds(r0 + PA, PA)], rows)
            pltpu.sync_copy(bat_hbm.at[pl.ds(r0 + PA, PA)], bat.at[pl.ds(0, PA)])
            _pool_rows(rows, bat, mx, sm, cn, lane0, PA)

        @pl.when(wid == NC * NS - 1)
        def _():
            pltpu.sync_copy(h_hbm.at[pl.ds(r0 + PA, PB_LAST)],
                            rows.at[pl.ds(0, PB_LAST)])
            pltpu.sync_copy(bat_hbm.at[pl.ds(r0 + PA, PB_LAST)],
                            bat.at[pl.ds(0, PB_LAST)])
            _pool_rows(rows, bat, mx, sm, cn, lane0, PB_LAST)

        pltpu.sync_copy(mx, omax.at[wid])
        pltpu.sync_copy(sm, osum.at[wid])
        pltpu.sync_copy(cn, ocnt.at[wid])

    return pool


_pool = _make_pool()


R = 2000
NB = N // R


def _make_layer_body(combine):
    def _ya_body(s_ref, c_ref, hp_ref, w_ref, bl_ref, y_ref, st_ref):
        i = pl.program_id(0)
        s = s_ref[...]
        if combine == "sum":
            s = s[:, :16] + s[:, 16:]
            c = s[:, 6:7]
        else:
            c = c_ref[...]
        inv = 1.0 / jnp.maximum(c, 1.0)
        z = jnp.concatenate([s * inv, hp_ref[...]], axis=1)
        y = jnp.dot(z, w_ref[...], preferred_element_type=_F32) + bl_ref[...]
        y_ref[...] = y

        @pl.when(i == 0)
        def _():
            st_ref[...] = jnp.zeros((8, H), _F32)

        su = jnp.sum(y, axis=0)
        sq = jnp.sum(y * y, axis=0)
        upd = jnp.concatenate(
            [su[None, :], sq[None, :], jnp.zeros((6, H), _F32)], axis=0)
        st_ref[...] = st_ref[...] + upd

    return _ya_body


def _layer_a(s, cnt, hp, Wl, bl, Wr, combine):
    K = s.shape[1]
    K2 = hp.shape[1]
    w = jnp.concatenate([Wl, Wr], axis=0)
    return pl.pallas_call(
        _make_layer_body(combine),
        grid=(NB,),
        in_specs=[
            pl.BlockSpec((R, K), lambda i: (i, 0)),
            pl.BlockSpec((R, 1), lambda i: (i, 0)),
            pl.BlockSpec((R, K2), lambda i: (i, 0)),
            pl.BlockSpec((w.shape[0], H), lambda i: (0, 0)),
            pl.BlockSpec((1, H), lambda i: (0, 0)),
        ],
        out_specs=[
            pl.BlockSpec((R, H), lambda i: (i, 0)),
            pl.BlockSpec((8, H), lambda i: (0, 0)),
        ],
        out_shape=[
            jax.ShapeDtypeStruct((N, H), _F32),
            jax.ShapeDtypeStruct((8, H), _F32),
        ],
    )(s, cnt, hp, w, bl)


def _yb_body(y_ref, st_ref, g_ref, be_ref, h_ref):
    st = st_ref[...]
    mean = st[0:1, :] * (1.0 / N)
    ex2 = st[1:2, :] * (1.0 / N)
    var = ex2 - mean * mean
    rstd = lax.rsqrt(var + 1e-5)
    h_ref[...] = jnp.maximum(
        (y_ref[...] - mean) * (rstd * g_ref[...]) + be_ref[...], 0.0)


def _layer_b(y, st, g, be):
    return pl.pallas_call(
        _yb_body,
        grid=(NB,),
        in_specs=[
            pl.BlockSpec((R, H), lambda i: (i, 0)),
            pl.BlockSpec((8, H), lambda i: (0, 0)),
            pl.BlockSpec((1, H), lambda i: (0, 0)),
            pl.BlockSpec((1, H), lambda i: (0, 0)),
        ],
        out_specs=pl.BlockSpec((R, H), lambda i: (i, 0)),
        out_shape=jax.ShapeDtypeStruct((N, H), _F32),
    )(y, st, g, be)


def _head_body(pm_ref, ps_ref, pc_ref, w1_ref, b1_ref, w2_ref, b2_ref, o_ref):
    mx = jnp.max(pm_ref[...], axis=0)
    sm = jnp.sum(ps_ref[...], axis=0)
    cnt = jnp.sum(pc_ref[...], axis=0)[:, 0:1]
    mean = sm / jnp.maximum(cnt, 1.0)
    z = jnp.concatenate([mx, mean], axis=1)
    r = jnp.maximum(
        jnp.dot(z, w1_ref[...], preferred_element_type=_F32) + b1_ref[...], 0.0)
    o_ref[...] = jnp.dot(r, w2_ref[...], preferred_element_type=_F32) + b2_ref[...]


def _head(pmax, psum, pcnt, W1, b1, W2, b2):
    return pl.pallas_call(
        _head_body,
        out_shape=jax.ShapeDtypeStruct((G, 2), _F32),
    )(pmax, psum, pcnt, W1, b1, W2, b2)


def kernel(x, edge_index, batch, Wl1, bl1, Wr1, g1, be1, Wl2, bl2, Wr2, g2,
           be2, Wl3, bl3, Wr3, g3, be3, W_lin1, b_lin1, W_lin2, b_lin2):
    src = edge_index[0]
    dst = edge_index[1]
    dst2 = dst.reshape(E // CB, CB)

    x16 = jnp.concatenate(
        [x, jnp.ones((N, 1), _F32), jnp.zeros((N, 9), _F32)], axis=1)
    Wl1p = jnp.zeros((16, H), _F32).at[:6].set(Wl1)
    Wr1p = jnp.zeros((16, H), _F32).at[:6].set(Wr1)

    s1 = _segsum16(x16, src, dst2)
    cnt = s1[:, 6:7] + s1[:, 22:23]

    y1, st1 = _layer_a(s1, cnt, x16, Wl1p, bl1.reshape(1, H), Wr1p, "sum")
    h1 = _layer_b(y1, st1, g1.reshape(1, H), be1.reshape(1, H))

    s2 = _segsum32(h1.reshape(2 * N, H // 2), src, dst2)
    y2, st2 = _layer_a(s2, cnt, h1, Wl2, bl2.reshape(1, H), Wr2, "cat")
    h2 = _layer_b(y2, st2, g2.reshape(1, H), be2.reshape(1, H))

    s3 = _segsum32(h2.reshape(2 * N, H // 2), src, dst2)
    y3, st3 = _layer_a(s3, cnt, h2, Wl3, bl3.reshape(1, H), Wr3, "cat")
    h3 = _layer_b(y3, st3, g3.reshape(1, H), be3.reshape(1, H))

    pmax, psum, pcnt = _pool(h3, batch)
    out = _head(pmax, psum, pcnt, W_lin1, b_lin1.reshape(1, H),
                W_lin2, b_lin2.reshape(1, 2))
    return out


# speedup vs baseline: 1.2275x; 1.0556x over previous
import functools

import jax
import jax.numpy as jnp
from jax import lax
from jax.experimental import pallas as pl
from jax.experimental.pallas import tpu as pltpu
from jax.experimental.pallas import tpu_sc as plsc

N = 50000
E = 800000
H = 64
G = 128

NC = 2
NS = 16

PADN = 50048
STRIPE = PADN // NS
WB_LAST = N - (NS - 1) * STRIPE

CB = 80
GQ = 5
GE = GQ * CB
ZB = 392

_F32 = jnp.float32
_I32 = jnp.int32


def _make_segsum(D, mode):
    mesh = plsc.VectorSubcoreMesh(core_axis_name="c", subcore_axis_name="s")
    nchunk = (E if mode == "cols" else E // 2) // CB
    ngrp_sc = nchunk // GQ

    @functools.partial(
        pl.kernel,
        mesh=mesh,
        out_type=jax.ShapeDtypeStruct((N, NC * D), _F32),
        scratch_types=[
            pltpu.VMEM((2, GE), _I32),
            pltpu.VMEM((2 * GQ, CB), _I32),
            [pltpu.VMEM((CB,), _I32) for _ in range(GQ)],
            [pltpu.VMEM((CB, D), _F32) for _ in range(GQ)],
            pltpu.VMEM((ZB, D), _F32),
            pltpu.VMEM_SHARED((PADN, D), _F32),
            [pltpu.SemaphoreType.DMA for _ in range(GQ)],
            pltpu.SemaphoreType.DMA,
            pltpu.SemaphoreType.DMA,
        ],
        compiler_params=pltpu.CompilerParams(use_tc_tiling_on_sc=False),
    )
    def seg(h_hbm, src_hbm, dst2_hbm, out_hbm, sg, dg, gl, rows, zb, acc,
            gsem, ssem, isem):
        cid = lax.axis_index("c")
        sid = lax.axis_index("s")

        z16 = jnp.zeros((16,), _F32)

        def zfill(i, carry):
            for t in range(D // 16):
                zb[i, pl.ds(t * 16, 16)] = z16
            return carry

        lax.fori_loop(0, ZB, zfill, 0)
        zr0 = sid * STRIPE
        for q in range(STRIPE // ZB):
            pltpu.sync_copy(zb, acc.at[pl.ds(zr0 + q * ZB, ZB)])
        zrem = STRIPE % ZB
        if zrem:
            pltpu.sync_copy(zb.at[pl.ds(0, zrem)],
                            acc.at[pl.ds(zr0 + (STRIPE // ZB) * ZB, zrem)])
        plsc.subcore_barrier()

        if mode == "cols":
            ngrp_t = ngrp_sc // NS
            grp0 = sid * ngrp_t
            gstep = 1
        else:
            ngrp_t = (ngrp_sc - sid + NS - 1) // NS
            grp0 = cid * ngrp_sc + sid
            gstep = NS

        def fetch_idx(j, parity):
            cbase = (grp0 + j * gstep) * GQ
            pltpu.async_copy(src_hbm.at[pl.ds(cbase * CB, GE)], sg.at[parity],
                             isem)
            pltpu.async_copy(dst2_hbm.at[pl.ds(cbase, GQ)],
                             dg.at[pl.ds(parity * GQ, GQ)], isem)

        fetch_idx(0, 0)

        def group(j, carry):
            par = lax.rem(j, 2)
            pltpu.make_async_copy(src_hbm.at[pl.ds(0, GE)], sg.at[par],
                                  isem).wait()
            pltpu.make_async_copy(dst2_hbm.at[pl.ds(0, GQ)],
                                  dg.at[pl.ds(0, GQ)], isem).wait()
            @pl.when(j > 0)
            def _():
                for q in range(GQ):
                    pltpu.make_async_copy(
                        h_hbm.at[pl.ds(0, CB)], rows[q], ssem).wait()

            @pl.when(j + 1 < ngrp_t)
            def _():
                fetch_idx(j + 1, 1 - par)

            gd = []
            for q in range(GQ):
                for t in range(CB // 16):
                    sv = sg[par, pl.ds(q * CB + t * 16, 16)]
                    if mode == "cols":
                        gl[q][pl.ds(t * 16, 16)] = sv * 2 + cid
                    else:
                        gl[q][pl.ds(t * 16, 16)] = sv
                gd.append(pltpu.async_copy(h_hbm.at[gl[q]], rows[q], gsem[q]))
            for q in range(GQ):
                gd[q].wait()
                pltpu.async_copy(rows[q], acc.at[dg.at[par * GQ + q]], ssem,
                                 add=True)
            return carry

        lax.fori_loop(0, ngrp_t, group, 0)
        for q in range(GQ):
            pltpu.make_async_copy(h_hbm.at[pl.ds(0, CB)], rows[q], ssem).wait()
        plsc.subcore_barrier()

        r0 = sid * STRIPE

        @pl.when(sid < NS - 1)
        def _():
            pltpu.sync_copy(acc.at[pl.ds(r0, STRIPE)],
                            out_hbm.at[pl.ds(r0, STRIPE),
                                       pl.ds(cid * D, D)])

        @pl.when(sid == NS - 1)
        def _():
            pltpu.sync_copy(acc.at[pl.ds(r0, WB_LAST)],
                            out_hbm.at[pl.ds(r0, WB_LAST),
                                       pl.ds(cid * D, D)])

    return seg


_segsum16 = _make_segsum(16, "edges")
_segsum32 = _make_segsum(32, "cols")


PR = 1568
PA = 784
PB_LAST = N - 31 * PR - PA


def _pool_rows(rows, bat, mx, sm, cn, lane0, nrows):
    def row(r, carry):
        seg = bat[pl.ds(r, 16)][0]
        for t in range(4):
            v = rows[r, pl.ds(t * 16, 16)]
            mx[seg, pl.ds(t * 16, 16)] = jnp.maximum(mx[seg, pl.ds(t * 16, 16)], v)
            sm[seg, pl.ds(t * 16, 16)] = sm[seg, pl.ds(t * 16, 16)] + v
        cn[seg, pl.ds(0, 16)] = cn[seg, pl.ds(0, 16)] + lane0
        return carry

    lax.fori_loop(0, nrows, row, 0)


def _make_pool():
    mesh = plsc.VectorSubcoreMesh(core_axis_name="c", subcore_axis_name="s")

    @functools.partial(
        pl.kernel,
        mesh=mesh,
        out_type=(
            jax.ShapeDtypeStruct((NC * NS, G, H), _F32),
            jax.ShapeDtypeStruct((NC * NS, G, H), _F32),
            jax.ShapeDtypeStruct((NC * NS, G, 16), _F32),
        ),
        scratch_types=[
            pltpu.VMEM((PA, H), _F32),
            pltpu.VMEM((PA + 16,), _I32),
            pltpu.VMEM((G, H), _F32),
            pltpu.VMEM((G, H), _F32),
            pltpu.VMEM((G, 16), _F32),
        ],
        compiler_params=pltpu.CompilerParams(use_tc_tiling_on_sc=False),
    )
    def pool(h_hbm, bat_hbm, omax, osum, ocnt, rows, bat, mx, sm, cn):
        cid = lax.axis_index("c")
        sid = lax.axis_index("s")
        wid = sid * NC + cid
        r0 = wid * PR

        ninf = jnp.full((16,), -jnp.inf, _F32)
        z16 = jnp.zeros((16,), _F32)
        iota = lax.iota(_I32, 16)
        lane0 = jnp.where(iota == 0, 1.0, 0.0).astype(_F32)

        def init(i, carry):
            for t in range(4):
                mx[i, pl.ds(t * 16, 16)] = ninf
                sm[i, pl.ds(t * 16, 16)] = z16
            cn[i, pl.ds(0, 16)] = z16
            return carry

        lax.fori_loop(0, G, init, 0)

        pltpu.sync_copy(h_hbm.at[pl.ds(r0, PA)], rows)
        pltpu.sync_copy(bat_hbm.at[pl.ds(r0, PA)], bat.at[pl.ds(0, PA)])
        _pool_rows(rows, bat, mx, sm, cn, lane0, PA)

        @pl.when(wid < NC * NS - 1)
        def _():
            pltpu.sync_copy(h_hbm.at[pl.ds(r0 + PA, PA)], rows)
            pltpu.sync_copy(bat_hbm.at[pl.ds(r0 + PA, PA)], bat.at[pl.ds(0, PA)])
            _pool_rows(rows, bat, mx, sm, cn, lane0, PA)

        @pl.when(wid == NC * NS - 1)
        def _():
            pltpu.sync_copy(h_hbm.at[pl.ds(r0 + PA, PB_LAST)],
                            rows.at[pl.ds(0, PB_LAST)])
            pltpu.sync_copy(bat_hbm.at[pl.ds(r0 + PA, PB_LAST)],
                            bat.at[pl.ds(0, PB_LAST)])
            _pool_rows(rows, bat, mx, sm, cn, lane0, PB_LAST)

        pltpu.sync_copy(mx, omax.at[wid])
        pltpu.sync_copy(sm, osum.at[wid])
        pltpu.sync_copy(cn, ocnt.at[wid])

    return pool


_pool = _make_pool()


R = 2000
NB = N // R


def _make_layer_body(combine):
    def _body(s_ref, s1_ref, hp_ref, w_ref, bl_ref, g_ref, be_ref, h_ref,
              st_ref, y_ref):
        p = pl.program_id(0)
        i = pl.program_id(1)

        @pl.when(p == 0)
        def _():
            s = s_ref[...]
            if combine == "sum":
                s = s[:, :16] + s[:, 16:]
                c = s[:, 6:7]
            else:
                s1 = s1_ref[...]
                c = s1[:, 6:7] + s1[:, 22:23]
            inv = 1.0 / jnp.maximum(c, 1.0)
            z = jnp.concatenate([s * inv, hp_ref[...]], axis=1)
            y = (jnp.dot(z, w_ref[...], preferred_element_type=_F32)
                 + bl_ref[...])
            y_ref[pl.ds(i * R, R), :] = y

            @pl.when(i == 0)
            def _():
                st_ref[...] = jnp.zeros((8, H), _F32)

            su = jnp.sum(y, axis=0)
            sq = jnp.sum(y * y, axis=0)
            upd = jnp.concatenate(
                [su[None, :], sq[None, :], jnp.zeros((6, H), _F32)], axis=0)
            st_ref[...] = st_ref[...] + upd

        @pl.when(p == 1)
        def _():
            st = st_ref[...]
            mean = st[0:1, :] * (1.0 / N)
            ex2 = st[1:2, :] * (1.0 / N)
            var = ex2 - mean * mean
            rstd = lax.rsqrt(var + 1e-5)
            h_ref[...] = jnp.maximum(
                (y_ref[pl.ds(i * R, R), :] - mean) * (rstd * g_ref[...])
                + be_ref[...], 0.0)

    return _body


def _layer_tc(s, s1, hp, Wl, bl, Wr, g, be, combine):
    K = s.shape[1]
    K2 = hp.shape[1]
    w = jnp.concatenate([Wl, Wr], axis=0)
    return pl.pallas_call(
        _make_layer_body(combine),
        grid=(2, NB),
        in_specs=[
            pl.BlockSpec((R, K), lambda p, i: (i * (1 - p), 0)),
            pl.BlockSpec((R, 32), lambda p, i: (i * (1 - p), 0)),
            pl.BlockSpec((R, K2), lambda p, i: (i * (1 - p), 0)),
            pl.BlockSpec((w.shape[0], H), lambda p, i: (0, 0)),
            pl.BlockSpec((1, H), lambda p, i: (0, 0)),
            pl.BlockSpec((1, H), lambda p, i: (0, 0)),
            pl.BlockSpec((1, H), lambda p, i: (0, 0)),
        ],
        out_specs=pl.BlockSpec((R, H), lambda p, i: (i, 0)),
        out_shape=jax.ShapeDtypeStruct((N, H), _F32),
        scratch_shapes=[pltpu.VMEM((8, H), _F32), pltpu.VMEM((N, H), _F32)],
    )(s, s1, hp, w, bl, g, be)


def _head_body(pm_ref, ps_ref, pc_ref, w1_ref, b1_ref, w2_ref, b2_ref, o_ref):
    mx = jnp.max(pm_ref[...], axis=0)
    sm = jnp.sum(ps_ref[...], axis=0)
    cnt = jnp.sum(pc_ref[...], axis=0)[:, 0:1]
    mean = sm / jnp.maximum(cnt, 1.0)
    z = jnp.concatenate([mx, mean], axis=1)
    r = jnp.maximum(
        jnp.dot(z, w1_ref[...], preferred_element_type=_F32) + b1_ref[...], 0.0)
    o_ref[...] = jnp.dot(r, w2_ref[...], preferred_element_type=_F32) + b2_ref[...]


def _head(pmax, psum, pcnt, W1, b1, W2, b2):
    return pl.pallas_call(
        _head_body,
        out_shape=jax.ShapeDtypeStruct((G, 2), _F32),
    )(pmax, psum, pcnt, W1, b1, W2, b2)


def kernel(x, edge_index, batch, Wl1, bl1, Wr1, g1, be1, Wl2, bl2, Wr2, g2,
           be2, Wl3, bl3, Wr3, g3, be3, W_lin1, b_lin1, W_lin2, b_lin2):
    src = edge_index[0]
    dst = edge_index[1]
    dst2 = dst.reshape(E // CB, CB)

    x16 = jnp.concatenate(
        [x, jnp.ones((N, 1), _F32), jnp.zeros((N, 9), _F32)], axis=1)
    Wl1p = jnp.zeros((16, H), _F32).at[:6].set(Wl1)
    Wr1p = jnp.zeros((16, H), _F32).at[:6].set(Wr1)

    s1 = _segsum16(x16, src, dst2)

    h1 = _layer_tc(s1, s1, x16, Wl1p, bl1.reshape(1, H), Wr1p,
                   g1.reshape(1, H), be1.reshape(1, H), "sum")

    s2 = _segsum32(h1.reshape(2 * N, H // 2), src, dst2)
    h2 = _layer_tc(s2, s1, h1, Wl2, bl2.reshape(1, H), Wr2,
                   g2.reshape(1, H), be2.reshape(1, H), "cat")

    s3 = _segsum32(h2.reshape(2 * N, H // 2), src, dst2)
    h3 = _layer_tc(s3, s1, h2, Wl3, bl3.reshape(1, H), Wr3,
                   g3.reshape(1, H), be3.reshape(1, H), "cat")

    pmax, psum, pcnt = _pool(h3, batch)
    out = _head(pmax, psum, pcnt, W_lin1, b_lin1.reshape(1, H),
                W_lin2, b_lin2.reshape(1, 2))
    return out


# speedup vs baseline: 1.2773x; 1.0406x over previous
import functools

import jax
import jax.numpy as jnp
from jax import lax
from jax.experimental import pallas as pl
from jax.experimental.pallas import tpu as pltpu
from jax.experimental.pallas import tpu_sc as plsc

N = 50000
E = 800000
H = 64
G = 128

NC = 2
NS = 16

PADN = 50048
STRIPE = PADN // NS
WB_LAST = N - (NS - 1) * STRIPE

CB = 80
GQ = 5
GE = GQ * CB
ZB = 392

_F32 = jnp.float32
_I32 = jnp.int32


def _make_segsum(D, mode):
    mesh = plsc.VectorSubcoreMesh(core_axis_name="c", subcore_axis_name="s")
    nchunk = (E if mode == "cols" else E // 2) // CB
    ngrp_sc = nchunk // GQ

    @functools.partial(
        pl.kernel,
        mesh=mesh,
        out_type=jax.ShapeDtypeStruct((N, NC * D), _F32),
        scratch_types=[
            pltpu.VMEM((2, GE), _I32),
            pltpu.VMEM((2 * GQ, CB), _I32),
            [pltpu.VMEM((CB,), _I32) for _ in range(GQ)],
            [pltpu.VMEM((CB, D), _F32) for _ in range(GQ)],
            pltpu.VMEM((ZB, D), _F32),
            pltpu.VMEM_SHARED((PADN, D), _F32),
            [pltpu.SemaphoreType.DMA for _ in range(GQ)],
            pltpu.SemaphoreType.DMA,
            pltpu.SemaphoreType.DMA,
        ],
        compiler_params=pltpu.CompilerParams(use_tc_tiling_on_sc=False),
    )
    def seg(h_hbm, src_hbm, dst2_hbm, out_hbm, sg, dg, gl, rows, zb, acc,
            gsem, ssem, isem):
        cid = lax.axis_index("c")
        sid = lax.axis_index("s")

        z16 = jnp.zeros((16,), _F32)

        def zfill(i, carry):
            for t in range(D // 16):
                zb[i, pl.ds(t * 16, 16)] = z16
            return carry

        lax.fori_loop(0, ZB, zfill, 0)
        zr0 = sid * STRIPE
        for q in range(STRIPE // ZB):
            pltpu.sync_copy(zb, acc.at[pl.ds(zr0 + q * ZB, ZB)])
        zrem = STRIPE % ZB
        if zrem:
            pltpu.sync_copy(zb.at[pl.ds(0, zrem)],
                            acc.at[pl.ds(zr0 + (STRIPE // ZB) * ZB, zrem)])
        plsc.subcore_barrier()

        if mode == "cols":
            ngrp_t = ngrp_sc // NS
            grp0 = sid * ngrp_t
            gstep = 1
        else:
            ngrp_t = (ngrp_sc - sid + NS - 1) // NS
            grp0 = cid * ngrp_sc + sid
            gstep = NS

        def fetch_idx(j, parity):
            cbase = (grp0 + j * gstep) * GQ
            pltpu.async_copy(src_hbm.at[pl.ds(cbase * CB, GE)], sg.at[parity],
                             isem)
            pltpu.async_copy(dst2_hbm.at[pl.ds(cbase, GQ)],
                             dg.at[pl.ds(parity * GQ, GQ)], isem)

        fetch_idx(0, 0)

        def group(j, carry):
            par = lax.rem(j, 2)
            pltpu.make_async_copy(src_hbm.at[pl.ds(0, GE)], sg.at[par],
                                  isem).wait()
            pltpu.make_async_copy(dst2_hbm.at[pl.ds(0, GQ)],
                                  dg.at[pl.ds(0, GQ)], isem).wait()
            @pl.when(j > 0)
            def _():
                for q in range(GQ):
                    pltpu.make_async_copy(
                        h_hbm.at[pl.ds(0, CB)], rows[q], ssem).wait()

            @pl.when(j + 1 < ngrp_t)
            def _():
                fetch_idx(j + 1, 1 - par)

            gd = []
            for q in range(GQ):
                for t in range(CB // 16):
                    sv = sg[par, pl.ds(q * CB + t * 16, 16)]
                    if mode == "cols":
                        gl[q][pl.ds(t * 16, 16)] = sv * 2 + cid
                    else:
                        gl[q][pl.ds(t * 16, 16)] = sv
                gd.append(pltpu.async_copy(h_hbm.at[gl[q]], rows[q], gsem[q]))
            for q in range(GQ):
                gd[q].wait()
                pltpu.async_copy(rows[q], acc.at[dg.at[par * GQ + q]], ssem,
                                 add=True)
            return carry

        lax.fori_loop(0, ngrp_t, group, 0)
        for q in range(GQ):
            pltpu.make_async_copy(h_hbm.at[pl.ds(0, CB)], rows[q], ssem).wait()
        plsc.subcore_barrier()

        r0 = sid * STRIPE

        @pl.when(sid < NS - 1)
        def _():
            pltpu.sync_copy(acc.at[pl.ds(r0, STRIPE)],
                            out_hbm.at[pl.ds(r0, STRIPE),
                                       pl.ds(cid * D, D)])

        @pl.when(sid == NS - 1)
        def _():
            pltpu.sync_copy(acc.at[pl.ds(r0, WB_LAST)],
                            out_hbm.at[pl.ds(r0, WB_LAST),
                                       pl.ds(cid * D, D)])

    return seg


_segsum16 = _make_segsum(16, "edges")
_segsum32 = _make_segsum(32, "cols")


PR = 1568
PA = 784
PB_LAST = N - 31 * PR - PA


def _pool_rows(rows, bat, mx, sm, cn, lane0, nrows):
    def row(r, carry):
        seg = bat[pl.ds(r, 16)][0]
        for t in range(4):
            v = rows[r, pl.ds(t * 16, 16)]
            mx[seg, pl.ds(t * 16, 16)] = jnp.maximum(mx[seg, pl.ds(t * 16, 16)], v)
            sm[seg, pl.ds(t * 16, 16)] = sm[seg, pl.ds(t * 16, 16)] + v
        cn[seg, pl.ds(0, 16)] = cn[seg, pl.ds(0, 16)] + lane0
        return carry

    lax.fori_loop(0, nrows, row, 0)


def _make_pool():
    mesh = plsc.VectorSubcoreMesh(core_axis_name="c", subcore_axis_name="s")

    @functools.partial(
        pl.kernel,
        mesh=mesh,
        out_type=(
            jax.ShapeDtypeStruct((NC * NS, G, H), _F32),
            jax.ShapeDtypeStruct((NC * NS, G, H), _F32),
            jax.ShapeDtypeStruct((NC * NS, G, 16), _F32),
        ),
        scratch_types=[
            pltpu.VMEM((PA, H), _F32),
            pltpu.VMEM((PA + 16,), _I32),
            pltpu.VMEM((G, H), _F32),
            pltpu.VMEM((G, H), _F32),
            pltpu.VMEM((G, 16), _F32),
        ],
        compiler_params=pltpu.CompilerParams(use_tc_tiling_on_sc=False),
    )
    def pool(h_hbm, bat_hbm, omax, osum, ocnt, rows, bat, mx, sm, cn):
        cid = lax.axis_index("c")
        sid = lax.axis_index("s")
        wid = sid * NC + cid
        r0 = wid * PR

        ninf = jnp.full((16,), -jnp.inf, _F32)
        z16 = jnp.zeros((16,), _F32)
        iota = lax.iota(_I32, 16)
        lane0 = jnp.where(iota == 0, 1.0, 0.0).astype(_F32)

        def init(i, carry):
            for t in range(4):
                mx[i, pl.ds(t * 16, 16)] = ninf
                sm[i, pl.ds(t * 16, 16)] = z16
            cn[i, pl.ds(0, 16)] = z16
            return carry

        lax.fori_loop(0, G, init, 0)

        pltpu.sync_copy(h_hbm.at[pl.ds(r0, PA)], rows)
        pltpu.sync_copy(bat_hbm.at[pl.ds(r0, PA)], bat.at[pl.ds(0, PA)])
        _pool_rows(rows, bat, mx, sm, cn, lane0, PA)

        @pl.when(wid < NC * NS - 1)
        def _():
            pltpu.sync_copy(h_hbm.at[pl.ds(r0 + PA, PA)], rows)
            pltpu.sync_copy(bat_hbm.at[pl.ds(r0 + PA, PA)], bat.at[pl.ds(0, PA)])
            _pool_rows(rows, bat, mx, sm, cn, lane0, PA)

        @pl.when(wid == NC * NS - 1)
        def _():
            pltpu.sync_copy(h_hbm.at[pl.ds(r0 + PA, PB_LAST)],
                            rows.at[pl.ds(0, PB_LAST)])
            pltpu.sync_copy(bat_hbm.at[pl.ds(r0 + PA, PB_LAST)],
                            bat.at[pl.ds(0, PB_LAST)])
            _pool_rows(rows, bat, mx, sm, cn, lane0, PB_LAST)

        pltpu.sync_copy(mx, omax.at[wid])
        pltpu.sync_copy(sm, osum.at[wid])
        pltpu.sync_copy(cn, ocnt.at[wid])

    return pool


_pool = _make_pool()


R = 5000
NB = N // R


def _make_layer_body(combine):
    def _body(s_ref, s1_ref, hp_ref, w_ref, bl_ref, g_ref, be_ref, h_ref,
              st_ref, y_ref):
        p = pl.program_id(0)
        i = pl.program_id(1)

        @pl.when(p == 0)
        def _():
            s = s_ref[...]
            if combine == "sum":
                s = s[:, :16] + s[:, 16:]
                c = s[:, 6:7]
            else:
                s1 = s1_ref[...]
                c = s1[:, 6:7] + s1[:, 22:23]
            inv = 1.0 / jnp.maximum(c, 1.0)
            z = jnp.concatenate([s * inv, hp_ref[...]], axis=1)
            y = (jnp.dot(z, w_ref[...], preferred_element_type=_F32)
                 + bl_ref[...])
            y_ref[pl.ds(i * R, R), :] = y

            @pl.when(i == 0)
            def _():
                st_ref[...] = jnp.zeros((8, H), _F32)

            su = jnp.sum(y, axis=0)
            sq = jnp.sum(y * y, axis=0)
            upd = jnp.concatenate(
                [su[None, :], sq[None, :], jnp.zeros((6, H), _F32)], axis=0)
            st_ref[...] = st_ref[...] + upd

        @pl.when(p == 1)
        def _():
            st = st_ref[...]
            mean = st[0:1, :] * (1.0 / N)
            ex2 = st[1:2, :] * (1.0 / N)
            var = ex2 - mean * mean
            rstd = lax.rsqrt(var + 1e-5)
            h_ref[...] = jnp.maximum(
                (y_ref[pl.ds(i * R, R), :] - mean) * (rstd * g_ref[...])
                + be_ref[...], 0.0)

    return _body


def _layer_tc(s, s1, hp, Wl, bl, Wr, g, be, combine):
    K = s.shape[1]
    K2 = hp.shape[1]
    w = jnp.concatenate([Wl, Wr], axis=0)
    return pl.pallas_call(
        _make_layer_body(combine),
        grid=(2, NB),
        in_specs=[
            pl.BlockSpec((R, K), lambda p, i: (i * (1 - p), 0)),
            pl.BlockSpec((R, 32), lambda p, i: (i * (1 - p), 0)),
            pl.BlockSpec((R, K2), lambda p, i: (i * (1 - p), 0)),
            pl.BlockSpec((w.shape[0], H), lambda p, i: (0, 0)),
            pl.BlockSpec((1, H), lambda p, i: (0, 0)),
            pl.BlockSpec((1, H), lambda p, i: (0, 0)),
            pl.BlockSpec((1, H), lambda p, i: (0, 0)),
        ],
        out_specs=pl.BlockSpec((R, H), lambda p, i: (i, 0)),
        out_shape=jax.ShapeDtypeStruct((N, H), _F32),
        scratch_shapes=[pltpu.VMEM((8, H), _F32), pltpu.VMEM((N, H), _F32)],
    )(s, s1, hp, w, bl, g, be)


def _head_body(pm_ref, ps_ref, pc_ref, w1_ref, b1_ref, w2_ref, b2_ref, o_ref):
    mx = jnp.max(pm_ref[...], axis=0)
    sm = jnp.sum(ps_ref[...], axis=0)
    cnt = jnp.sum(pc_ref[...], axis=0)[:, 0:1]
    mean = sm / jnp.maximum(cnt, 1.0)
    z = jnp.concatenate([mx, mean], axis=1)
    r = jnp.maximum(
        jnp.dot(z, w1_ref[...], preferred_element_type=_F32) + b1_ref[...], 0.0)
    o_ref[...] = jnp.dot(r, w2_ref[...], preferred_element_type=_F32) + b2_ref[...]


def _head(pmax, psum, pcnt, W1, b1, W2, b2):
    return pl.pallas_call(
        _head_body,
        out_shape=jax.ShapeDtypeStruct((G, 2), _F32),
    )(pmax, psum, pcnt, W1, b1, W2, b2)


def kernel(x, edge_index, batch, Wl1, bl1, Wr1, g1, be1, Wl2, bl2, Wr2, g2,
           be2, Wl3, bl3, Wr3, g3, be3, W_lin1, b_lin1, W_lin2, b_lin2):
    src = edge_index[0]
    dst = edge_index[1]
    dst2 = dst.reshape(E // CB, CB)

    x16 = jnp.concatenate(
        [x, jnp.ones((N, 1), _F32), jnp.zeros((N, 9), _F32)], axis=1)
    Wl1p = jnp.zeros((16, H), _F32).at[:6].set(Wl1)
    Wr1p = jnp.zeros((16, H), _F32).at[:6].set(Wr1)

    s1 = _segsum16(x16, src, dst2)

    h1 = _layer_tc(s1, s1, x16, Wl1p, bl1.reshape(1, H), Wr1p,
                   g1.reshape(1, H), be1.reshape(1, H), "sum")

    s2 = _segsum32(h1.reshape(2 * N, H // 2), src, dst2)
    h2 = _layer_tc(s2, s1, h1, Wl2, bl2.reshape(1, H), Wr2,
                   g2.reshape(1, H), be2.reshape(1, H), "cat")

    s3 = _segsum32(h2.reshape(2 * N, H // 2), src, dst2)
    h3 = _layer_tc(s3, s1, h2, Wl3, bl3.reshape(1, H), Wr3,
                   g3.reshape(1, H), be3.reshape(1, H), "cat")

    pmax, psum, pcnt = _pool(h3, batch)
    out = _head(pmax, psum, pcnt, W_lin1, b_lin1.reshape(1, H),
                W_lin2, b_lin2.reshape(1, 2))
    return out
